# Initial kernel scaffold; baseline (speedup 1.0000x reference)
import jax
import jax.numpy as jnp
from jax.experimental import pallas as pl


def kernel(node, edge, edge_index, W_lin, b_lin, W_att, b_att, a_vec):
    raise NotImplementedError("write your pallas kernel here")


# speedup vs baseline: 10.4855x; 10.4855x over previous
import functools

import jax
import jax.numpy as jnp
from jax import lax
from jax.experimental import pallas as pl
from jax.experimental.pallas import tpu as pltpu
from jax.experimental.pallas import tpu_sc as plsc

F32 = jnp.float32

N = 10000
E = 320000
F = 128
U = 128

NC = 2
NS = 16
NW = NC * NS
EPW = E // NW
C = 80
NCHUNK = EPW // C
NP = 10240
DR = NP // 128
RPT = NP // NS
ZR = 128
KV = U // 16


def _tc_pre_body(node_ref, w_ref, b_ref, wn_ref, sin_ref, sout_ref):
    y = jnp.dot(node_ref[...], w_ref[...], preferred_element_type=F32)
    y = y + b_ref[...]
    wn_ref[...] = y[:, 0:U]
    sin_ref[...] = y[:, U:2 * U]
    sout_ref[...] = y[:, 2 * U:3 * U]


def _tc_pre(node, w_cat, b_cat):
    bn = 1000
    grid = (N // bn,)
    out = jax.ShapeDtypeStruct((N, U), F32)
    return pl.pallas_call(
        _tc_pre_body,
        grid=grid,
        in_specs=[
            pl.BlockSpec((bn, F), lambda i: (i, 0)),
            pl.BlockSpec((F, 3 * U), lambda i: (0, 0)),
            pl.BlockSpec((1, 3 * U), lambda i: (0, 0)),
        ],
        out_specs=[
            pl.BlockSpec((bn, U), lambda i: (i, 0)),
            pl.BlockSpec((bn, U), lambda i: (i, 0)),
            pl.BlockSpec((bn, U), lambda i: (i, 0)),
        ],
        out_shape=[out, out, out],
    )(node, w_cat, b_cat)


def _sc_body(sin_hbm, sout_hbm, wn_hbm, ii_hbm, io_hbm, av_hbm,
             num_out, den_out,
             av_v, ii_v, io_v, sin_v, sout_v, w_v, ex_v, dz_v, zb_v,
             acc_sh, den_sh, sem1, sem2, sem3):
    cid = lax.axis_index("c")
    sid = lax.axis_index("s")
    wid = cid * NS + sid
    z16 = jnp.zeros((16,), F32)
    iota16 = lax.iota(jnp.int32, 16)
    perms = [lax.bitwise_and(iota16 + sh, 15) for sh in (8, 4, 2, 1)]

    def _hsum(x):
        for p in perms:
            x = x + jnp.take_along_axis(x, p, axis=0)
        return x

    def _zden(r, _):
        dz_v[pl.ds(r * 16, 16)] = z16
        return _
    lax.fori_loop(0, RPT // 16, _zden, None)
    pltpu.sync_copy(dz_v, den_sh.at[pl.ds(sid * RPT, RPT)])

    def _zzb(r, _):
        for k in range(KV):
            zb_v[r, pl.ds(k * 16, 16)] = z16
        return _
    lax.fori_loop(0, ZR, _zzb, None)
    for j in range(RPT // ZR):
        pltpu.sync_copy(zb_v, acc_sh.at[pl.ds(sid * RPT + j * ZR, ZR)])
    plsc.subcore_barrier()

    pltpu.sync_copy(av_hbm, av_v)
    av = [av_v[pl.ds(k * 16, 16)] for k in range(KV)]

    def _chunk(t, _):
        off = wid * EPW + t * C
        pltpu.sync_copy(ii_hbm.at[pl.ds(off, C)], ii_v)
        pltpu.sync_copy(io_hbm.at[pl.ds(off, C)], io_v)
        cp1 = pltpu.async_copy(sin_hbm.at[ii_v], sin_v, sem1)
        cp2 = pltpu.async_copy(sout_hbm.at[io_v], sout_v, sem2)
        cp3 = pltpu.async_copy(wn_hbm.at[io_v], w_v, sem3)
        cp1.wait()
        cp2.wait()
        cp3.wait()

        def _group(g, _c):
            e0 = g * 16
            gv = z16
            for l in range(16):
                e = e0 + l
                acc = z16
                for k in range(KV):
                    tt = (sin_v[e, pl.ds(k * 16, 16)]
                          + sout_v[e, pl.ds(k * 16, 16)])
                    lr = jnp.maximum(tt, 0.2 * tt)
                    acc = acc + lr * av[k]
                gv = jnp.where(iota16 == l, _hsum(acc), gv)
            ex_v[pl.ds(e0, 16)] = jnp.exp(gv)
            return _c
        lax.fori_loop(0, C // 16, _group, None)

        def _scaleg(g, _c):
            e0 = g * 16
            exg = ex_v[pl.ds(e0, 16)]
            for l in range(16):
                e = e0 + l
                ex = exg[l]
                for k in range(KV):
                    w_v[e, pl.ds(k * 16, 16)] = w_v[e, pl.ds(k * 16, 16)] * ex
            return _c
        lax.fori_loop(0, C // 16, _scaleg, None)

        pltpu.sync_copy(w_v, acc_sh.at[ii_v], add=True)
        pltpu.sync_copy(ex_v, den_sh.at[ii_v], add=True)
        return _
    lax.fori_loop(0, NCHUNK, _chunk, None)

    plsc.subcore_barrier()

    for j in range(RPT // ZR):
        r0 = sid * RPT + j * ZR
        pltpu.sync_copy(acc_sh.at[pl.ds(r0, ZR)], num_out.at[cid, pl.ds(r0, ZR)])
    pltpu.sync_copy(den_sh.at[pl.ds(sid * RPT, RPT)],
                    den_out.at[pl.ds(cid * NP + sid * RPT, RPT)])


def _sc_call(sin, sout, wn, ii, io, av):
    mesh = plsc.VectorSubcoreMesh(core_axis_name="c", subcore_axis_name="s")
    kern = pl.kernel(
        _sc_body,
        out_type=(
            jax.ShapeDtypeStruct((NC, NP, U), F32),
            jax.ShapeDtypeStruct((NC * NP,), F32),
        ),
        mesh=mesh,
        scratch_types=[
            pltpu.VMEM((U,), F32),
            pltpu.VMEM((C,), jnp.int32),
            pltpu.VMEM((C,), jnp.int32),
            pltpu.VMEM((C, U), F32),
            pltpu.VMEM((C, U), F32),
            pltpu.VMEM((C, U), F32),
            pltpu.VMEM((C,), F32),
            pltpu.VMEM((RPT,), F32),
            pltpu.VMEM((ZR, U), F32),
            pltpu.VMEM_SHARED((NP, U), F32),
            pltpu.VMEM_SHARED((NP,), F32),
            pltpu.SemaphoreType.DMA,
            pltpu.SemaphoreType.DMA,
            pltpu.SemaphoreType.DMA,
        ],
    )
    return kern(sin, sout, wn, ii, io, av)


def _tc_post_body(np_ref, dp_ref, out_ref):
    num = np_ref[0] + np_ref[1]
    den = dp_ref[0] + dp_ref[1]
    den = jnp.where(den > 0.0, den, 1.0)
    h = num[:N] / den
    out_ref[...] = jnp.maximum(h, 0.2 * h)


def _tc_post(num_parts, den_flat):
    return pl.pallas_call(
        _tc_post_body,
        out_shape=jax.ShapeDtypeStruct((N, U), F32),
    )(num_parts, den_flat)


def kernel(node, edge, edge_index, W_lin, b_lin, W_att, b_att, a_vec):
    del edge
    w_cat = jnp.concatenate([W_lin, W_att[:F], W_att[F:]], axis=1)
    b_cat = jnp.concatenate(
        [b_lin, b_att, jnp.zeros((U,), F32)]).reshape(1, 3 * U)
    wn, sin, sout = _tc_pre(node, w_cat, b_cat)
    ii = edge_index[0]
    io = edge_index[1]
    av = a_vec[:, 0]
    num_parts, den_parts = _sc_call(sin, sout, wn, ii, io, av)
    dp = den_parts.reshape(NC, NP)[:, :N, None]
    return _tc_post(num_parts, dp)


# speedup vs baseline: 12.4000x; 1.1826x over previous
import functools

import jax
import jax.numpy as jnp
from jax import lax
from jax.experimental import pallas as pl
from jax.experimental.pallas import tpu as pltpu
from jax.experimental.pallas import tpu_sc as plsc

F32 = jnp.float32

N = 10000
E = 320000
F = 128
U = 128

NC = 2
NS = 16
NW = NC * NS
EPW = E // NW
C = 80
NCHUNK = EPW // C
KB = 25
KC = KB * C
NP = 10240
DR = NP // 128
RPT = NP // NS
ZR = 128
KV = U // 16


def _tc_pre_body(node_ref, w_ref, b_ref, wn_ref, sin_ref, sout_ref):
    y = jnp.dot(node_ref[...], w_ref[...], preferred_element_type=F32)
    y = y + b_ref[...]
    wn_ref[...] = y[:, 0:U]
    sin_ref[...] = y[:, U:2 * U]
    sout_ref[...] = y[:, 2 * U:3 * U]


def _tc_pre(node, w_cat, b_cat):
    bn = 1000
    grid = (N // bn,)
    out = jax.ShapeDtypeStruct((N, U), F32)
    return pl.pallas_call(
        _tc_pre_body,
        grid=grid,
        in_specs=[
            pl.BlockSpec((bn, F), lambda i: (i, 0)),
            pl.BlockSpec((F, 3 * U), lambda i: (0, 0)),
            pl.BlockSpec((1, 3 * U), lambda i: (0, 0)),
        ],
        out_specs=[
            pl.BlockSpec((bn, U), lambda i: (i, 0)),
            pl.BlockSpec((bn, U), lambda i: (i, 0)),
            pl.BlockSpec((bn, U), lambda i: (i, 0)),
        ],
        out_shape=[out, out, out],
    )(node, w_cat, b_cat)


def _sc_body(sin_hbm, sout_hbm, wn_hbm, ii_hbm, io_hbm, av_hbm,
             num_out, den_out,
             av_v, iia_v, ioa_v, ii_g, io_g, sin_v, sout_v, w_v, ex_v, dz_v,
             acc_sh, den_sh, sem1, sem2, sem3):
    cid = lax.axis_index("c")
    sid = lax.axis_index("s")
    wid = cid * NS + sid
    z16 = jnp.zeros((16,), F32)
    iota16 = lax.iota(jnp.int32, 16)
    perms = [lax.bitwise_and(iota16 + sh, 15) for sh in (8, 4, 2, 1)]

    def _hsum(x):
        for p in perms:
            x = x + jnp.take_along_axis(x, p, axis=0)
        return x

    def _zden(r, _):
        dz_v[pl.ds(r * 16, 16)] = z16
        return _
    lax.fori_loop(0, RPT // 16, _zden, None)
    pltpu.sync_copy(dz_v, den_sh.at[pl.ds(sid * RPT, RPT)])

    def _zzb(r, _):
        for k in range(KV):
            sin_v[r, pl.ds(k * 16, 16)] = z16
        return _
    lax.fori_loop(0, C, _zzb, None)
    for j in range(RPT // C):
        pltpu.sync_copy(sin_v, acc_sh.at[pl.ds(sid * RPT + j * C, C)])
    plsc.subcore_barrier()

    pltpu.sync_copy(av_hbm, av_v)
    av = [av_v[pl.ds(k * 16, 16)] for k in range(KV)]

    def _chunk(t, _):
        r = lax.rem(t, KB)

        @pl.when(r == 0)
        def _():
            off = wid * EPW + t * C
            pltpu.sync_copy(ii_hbm.at[pl.ds(off, KC)], iia_v)
            pltpu.sync_copy(io_hbm.at[pl.ds(off, KC)], ioa_v)

        for g in range(C // 16):
            dst = pl.ds(g * 16, 16)
            srcsl = pl.ds(r * C + g * 16, 16)
            ii_g[dst] = iia_v[srcsl]
            io_g[dst] = ioa_v[srcsl]

        cps = pltpu.async_copy(sin_hbm.at[ii_g], sin_v, sem1)
        cpo = pltpu.async_copy(sout_hbm.at[io_g], sout_v, sem2)
        cpw = pltpu.async_copy(wn_hbm.at[io_g], w_v, sem3)
        cps.wait()
        cpo.wait()

        def _group(g, _c):
            e0 = g * 16
            gv = z16
            for l in range(16):
                e = e0 + l
                acc = z16
                for k in range(KV):
                    tt = (sin_v[e, pl.ds(k * 16, 16)]
                          + sout_v[e, pl.ds(k * 16, 16)])
                    lr = jnp.maximum(tt, 0.2 * tt)
                    acc = acc + lr * av[k]
                gv = jnp.where(iota16 == l, _hsum(acc), gv)
            ex_v[pl.ds(e0, 16)] = jnp.exp(gv)
            return _c
        lax.fori_loop(0, C // 16, _group, None)

        cpw.wait()

        def _scaleg(g, _c):
            e0 = g * 16
            exg = ex_v[pl.ds(e0, 16)]
            for l in range(16):
                e = e0 + l
                ex = exg[l]
                for k in range(KV):
                    w_v[e, pl.ds(k * 16, 16)] = w_v[e, pl.ds(k * 16, 16)] * ex
            return _c
        lax.fori_loop(0, C // 16, _scaleg, None)

        pltpu.sync_copy(w_v, acc_sh.at[ii_g], add=True)
        pltpu.sync_copy(ex_v, den_sh.at[ii_g], add=True)
        return _
    lax.fori_loop(0, NCHUNK, _chunk, None)

    plsc.subcore_barrier()

    for j in range(RPT // ZR):
        r0 = sid * RPT + j * ZR
        pltpu.sync_copy(acc_sh.at[pl.ds(r0, ZR)], num_out.at[cid, pl.ds(r0, ZR)])
    pltpu.sync_copy(den_sh.at[pl.ds(sid * RPT, RPT)],
                    den_out.at[pl.ds(cid * NP + sid * RPT, RPT)])


def _sc_call(sin, sout, wn, ii, io, av):
    mesh = plsc.VectorSubcoreMesh(core_axis_name="c", subcore_axis_name="s")
    kern = pl.kernel(
        _sc_body,
        out_type=(
            jax.ShapeDtypeStruct((NC, NP, U), F32),
            jax.ShapeDtypeStruct((NC * NP,), F32),
        ),
        mesh=mesh,
        scratch_types=[
            pltpu.VMEM((U,), F32),
            pltpu.VMEM((KC,), jnp.int32),
            pltpu.VMEM((KC,), jnp.int32),
            pltpu.VMEM((C,), jnp.int32),
            pltpu.VMEM((C,), jnp.int32),
            pltpu.VMEM((C, U), F32),
            pltpu.VMEM((C, U), F32),
            pltpu.VMEM((C, U), F32),
            pltpu.VMEM((C,), F32),
            pltpu.VMEM((RPT,), F32),
            pltpu.VMEM_SHARED((NP, U), F32),
            pltpu.VMEM_SHARED((NP,), F32),
            pltpu.SemaphoreType.DMA,
            pltpu.SemaphoreType.DMA,
            pltpu.SemaphoreType.DMA,
        ],
    )
    return kern(sin, sout, wn, ii, io, av)


def _tc_post_body(np_ref, dp_ref, out_ref):
    num = np_ref[0] + np_ref[1]
    den = dp_ref[0] + dp_ref[1]
    den = jnp.where(den > 0.0, den, 1.0)
    h = num[:N] / den
    out_ref[...] = jnp.maximum(h, 0.2 * h)


def _tc_post(num_parts, den_flat):
    return pl.pallas_call(
        _tc_post_body,
        out_shape=jax.ShapeDtypeStruct((N, U), F32),
    )(num_parts, den_flat)


def kernel(node, edge, edge_index, W_lin, b_lin, W_att, b_att, a_vec):
    del edge
    w_cat = jnp.concatenate([W_lin, W_att[:F], W_att[F:]], axis=1)
    b_cat = jnp.concatenate(
        [b_lin, b_att, jnp.zeros((U,), F32)]).reshape(1, 3 * U)
    wn, sin, sout = _tc_pre(node, w_cat, b_cat)
    ii = edge_index[0]
    io = edge_index[1]
    av = a_vec[:, 0]
    num_parts, den_parts = _sc_call(sin, sout, wn, ii, io, av)
    dp = den_parts.reshape(NC, NP)[:, :N, None]
    return _tc_post(num_parts, dp)


# speedup vs baseline: 16.0197x; 1.2919x over previous
import functools

import jax
import jax.numpy as jnp
from jax import lax
from jax.experimental import pallas as pl
from jax.experimental.pallas import tpu as pltpu
from jax.experimental.pallas import tpu_sc as plsc

F32 = jnp.float32

N = 10000
E = 320000
F = 128
U = 128

NC = 2
NS = 16
NW = NC * NS
EPW = E // NW
H = 48
NH = 208
TAIL = EPW - NH * H
HG = H // 16
NP = 10240
DR = NP // 128
RPT = NP // NS
ZR = 128
KV = U // 16


def _tc_pre_body(node_ref, w_ref, b_ref, wn_ref, sin_ref, sout_ref):
    y = jnp.dot(node_ref[...], w_ref[...], preferred_element_type=F32)
    y = y + b_ref[...]
    wn_ref[...] = y[:, 0:U]
    sin_ref[...] = y[:, U:2 * U]
    sout_ref[...] = y[:, 2 * U:3 * U]


def _tc_pre(node, w_cat, b_cat):
    bn = 1000
    grid = (N // bn,)
    out = jax.ShapeDtypeStruct((N, U), F32)
    return pl.pallas_call(
        _tc_pre_body,
        grid=grid,
        in_specs=[
            pl.BlockSpec((bn, F), lambda i: (i, 0)),
            pl.BlockSpec((F, 3 * U), lambda i: (0, 0)),
            pl.BlockSpec((1, 3 * U), lambda i: (0, 0)),
        ],
        out_specs=[
            pl.BlockSpec((bn, U), lambda i: (i, 0)),
            pl.BlockSpec((bn, U), lambda i: (i, 0)),
            pl.BlockSpec((bn, U), lambda i: (i, 0)),
        ],
        out_shape=[out, out, out],
    )(node, w_cat, b_cat)


def _sc_body(sin_hbm, sout_hbm, wn_hbm, ii_hbm, io_hbm, av_hbm,
             num_out, den_out,
             av_v, ii0, ii1, io0, io1, iit, iot, sin0, sin1, sout0, sout1,
             w0, w1, ex0, ex1, dz_v,
             acc_sh, den_sh, is0, is1, gs0, gs1, gs2, gs3, gs4, gs5):
    ii_s = [ii0, ii1]
    io_s = [io0, io1]
    sin_s = [sin0, sin1]
    sout_s = [sout0, sout1]
    w_s = [w0, w1]
    ex_s = [ex0, ex1]
    isem = [is0, is1]
    gsem = [[gs0, gs1, gs2], [gs3, gs4, gs5]]
    cid = lax.axis_index("c")
    sid = lax.axis_index("s")
    wid = cid * NS + sid
    z16 = jnp.zeros((16,), F32)
    iota16 = lax.iota(jnp.int32, 16)
    perms = [lax.bitwise_and(iota16 + sh, 15) for sh in (8, 4, 2, 1)]

    def _hsum(x):
        for p in perms:
            x = x + jnp.take_along_axis(x, p, axis=0)
        return x

    def _zden(r, _):
        dz_v[pl.ds(r * 16, 16)] = z16
        return _
    lax.fori_loop(0, RPT // 16, _zden, None)
    pltpu.sync_copy(dz_v, den_sh.at[pl.ds(sid * RPT, RPT)])

    def _zzb(r, _):
        for k in range(KV):
            sin0[r, pl.ds(k * 16, 16)] = z16
            sin1[r, pl.ds(k * 16, 16)] = z16
        return _
    lax.fori_loop(0, H, _zzb, None)
    zoff = sid * RPT
    for j in range(RPT // (2 * H)):
        pltpu.sync_copy(sin0, acc_sh.at[pl.ds(zoff + j * 2 * H, H)])
        pltpu.sync_copy(sin1, acc_sh.at[pl.ds(zoff + j * 2 * H + H, H)])
    pltpu.sync_copy(sin0.at[pl.ds(0, 40)],
                    acc_sh.at[pl.ds(zoff + RPT - 64, 40)])
    pltpu.sync_copy(sin1.at[pl.ds(0, 24)],
                    acc_sh.at[pl.ds(zoff + RPT - 24, 24)])
    plsc.subcore_barrier()

    pltpu.sync_copy(av_hbm, av_v)
    av = [av_v[pl.ds(k * 16, 16)] for k in range(KV)]

    def _start_idx(h, s):
        off = wid * EPW + h * H
        pltpu.async_copy(ii_hbm.at[pl.ds(off, H)], ii_s[s], isem[s])
        pltpu.async_copy(io_hbm.at[pl.ds(off, H)], io_s[s], isem[s])

    def _wait_idx(h, s):
        off = wid * EPW + h * H
        pltpu.make_async_copy(ii_hbm.at[pl.ds(off, H)], ii_s[s],
                              isem[s]).wait()
        pltpu.make_async_copy(io_hbm.at[pl.ds(off, H)], io_s[s],
                              isem[s]).wait()

    def _start_gath(s):
        pltpu.async_copy(sin_hbm.at[ii_s[s]], sin_s[s], gsem[s][0])
        pltpu.async_copy(sout_hbm.at[io_s[s]], sout_s[s], gsem[s][1])
        pltpu.async_copy(wn_hbm.at[io_s[s]], w_s[s], gsem[s][2])

    def _wait_gath(s):
        pltpu.make_async_copy(sin_hbm.at[ii_s[s]], sin_s[s],
                              gsem[s][0]).wait()
        pltpu.make_async_copy(sout_hbm.at[io_s[s]], sout_s[s],
                              gsem[s][1]).wait()
        pltpu.make_async_copy(wn_hbm.at[io_s[s]], w_s[s], gsem[s][2]).wait()

    def _score(sin_v, sout_v, ex_v, ngroups):
        def _group(g, _c):
            e0 = g * 16
            gv = z16
            for l in range(16):
                e = e0 + l
                acc = z16
                for k in range(KV):
                    tt = (sin_v[e, pl.ds(k * 16, 16)]
                          + sout_v[e, pl.ds(k * 16, 16)])
                    lr = jnp.maximum(tt, 0.2 * tt)
                    acc = acc + lr * av[k]
                gv = jnp.where(iota16 == l, _hsum(acc), gv)
            ex_v[pl.ds(e0, 16)] = jnp.exp(gv)
            return _c
        lax.fori_loop(0, ngroups, _group, None)

    def _scale(w_v, ex_v, ngroups):
        def _scaleg(g, _c):
            e0 = g * 16
            exg = ex_v[pl.ds(e0, 16)]
            for l in range(16):
                e = e0 + l
                ex = exg[l]
                for k in range(KV):
                    w_v[e, pl.ds(k * 16, 16)] = w_v[e, pl.ds(k * 16, 16)] * ex
            return _c
        lax.fori_loop(0, ngroups, _scaleg, None)

    def _process(s):
        _score(sin_s[s], sout_s[s], ex_s[s], HG)
        _scale(w_s[s], ex_s[s], HG)
        pltpu.sync_copy(w_s[s], acc_sh.at[ii_s[s]], add=True)
        pltpu.sync_copy(ex_s[s], den_sh.at[ii_s[s]], add=True)

    _start_idx(0, 0)
    _wait_idx(0, 0)
    _start_gath(0)
    _start_idx(1, 1)

    def _pair(m, _):
        h0 = 2 * m
        _wait_idx(h0 + 1, 1)
        _wait_gath(0)
        _start_gath(1)
        _process(0)

        @pl.when(h0 + 2 < NH)
        def _():
            _start_idx(h0 + 2, 0)
            _wait_idx(h0 + 2, 0)
            _start_gath(0)
        _wait_gath(1)
        _process(1)

        @pl.when(h0 + 3 < NH)
        def _():
            _start_idx(h0 + 3, 1)
        return _
    lax.fori_loop(0, NH // 2, _pair, None)

    toff = wid * EPW + NH * H
    pltpu.sync_copy(ii_hbm.at[pl.ds(toff, TAIL)], iit)
    pltpu.sync_copy(io_hbm.at[pl.ds(toff, TAIL)], iot)
    cpt1 = pltpu.async_copy(sin_hbm.at[iit], sin0.at[pl.ds(0, TAIL)], gs0)
    cpt2 = pltpu.async_copy(sout_hbm.at[iot], sout0.at[pl.ds(0, TAIL)], gs1)
    cpt3 = pltpu.async_copy(wn_hbm.at[iot], w0.at[pl.ds(0, TAIL)], gs2)
    cpt1.wait()
    cpt2.wait()
    cpt3.wait()
    _score(sin0, sout0, ex0, TAIL // 16)
    _scale(w0, ex0, TAIL // 16)
    pltpu.sync_copy(w0.at[pl.ds(0, TAIL)], acc_sh.at[iit], add=True)
    pltpu.sync_copy(ex0.at[pl.ds(0, TAIL)], den_sh.at[iit], add=True)

    plsc.subcore_barrier()

    for j in range(RPT // ZR):
        r0 = sid * RPT + j * ZR
        pltpu.sync_copy(acc_sh.at[pl.ds(r0, ZR)], num_out.at[cid, pl.ds(r0, ZR)])
    pltpu.sync_copy(den_sh.at[pl.ds(sid * RPT, RPT)],
                    den_out.at[pl.ds(cid * NP + sid * RPT, RPT)])


def _sc_call(sin, sout, wn, ii, io, av):
    mesh = plsc.VectorSubcoreMesh(core_axis_name="c", subcore_axis_name="s")
    kern = pl.kernel(
        _sc_body,
        out_type=(
            jax.ShapeDtypeStruct((NC, NP, U), F32),
            jax.ShapeDtypeStruct((NC * NP,), F32),
        ),
        mesh=mesh,
        scratch_types=[
            pltpu.VMEM((U,), F32),
            pltpu.VMEM((H,), jnp.int32),
            pltpu.VMEM((H,), jnp.int32),
            pltpu.VMEM((H,), jnp.int32),
            pltpu.VMEM((H,), jnp.int32),
            pltpu.VMEM((TAIL,), jnp.int32),
            pltpu.VMEM((TAIL,), jnp.int32),
            pltpu.VMEM((H, U), F32),
            pltpu.VMEM((H, U), F32),
            pltpu.VMEM((H, U), F32),
            pltpu.VMEM((H, U), F32),
            pltpu.VMEM((H, U), F32),
            pltpu.VMEM((H, U), F32),
            pltpu.VMEM((H,), F32),
            pltpu.VMEM((H,), F32),
            pltpu.VMEM((RPT,), F32),
            pltpu.VMEM_SHARED((NP, U), F32),
            pltpu.VMEM_SHARED((NP,), F32),
            pltpu.SemaphoreType.DMA,
            pltpu.SemaphoreType.DMA,
            pltpu.SemaphoreType.DMA,
            pltpu.SemaphoreType.DMA,
            pltpu.SemaphoreType.DMA,
            pltpu.SemaphoreType.DMA,
            pltpu.SemaphoreType.DMA,
            pltpu.SemaphoreType.DMA,
        ],
    )
    return kern(sin, sout, wn, ii, io, av)


def _tc_post_body(np_ref, dp_ref, out_ref):
    num = np_ref[0] + np_ref[1]
    den = dp_ref[0] + dp_ref[1]
    den = jnp.where(den > 0.0, den, 1.0)
    h = num[:N] / den
    out_ref[...] = jnp.maximum(h, 0.2 * h)


def _tc_post(num_parts, den_flat):
    return pl.pallas_call(
        _tc_post_body,
        out_shape=jax.ShapeDtypeStruct((N, U), F32),
    )(num_parts, den_flat)


def kernel(node, edge, edge_index, W_lin, b_lin, W_att, b_att, a_vec):
    del edge
    w_cat = jnp.concatenate([W_lin, W_att[:F], W_att[F:]], axis=1)
    b_cat = jnp.concatenate(
        [b_lin, b_att, jnp.zeros((U,), F32)]).reshape(1, 3 * U)
    wn, sin, sout = _tc_pre(node, w_cat, b_cat)
    ii = edge_index[0]
    io = edge_index[1]
    av = a_vec[:, 0]
    num_parts, den_parts = _sc_call(sin, sout, wn, ii, io, av)
    dp = den_parts.reshape(NC, NP)[:, :N, None]
    return _tc_post(num_parts, dp)


# speedup vs baseline: 17.5548x; 1.0958x over previous
import functools

import jax
import jax.numpy as jnp
from jax import lax
from jax.experimental import pallas as pl
from jax.experimental.pallas import tpu as pltpu
from jax.experimental.pallas import tpu_sc as plsc

F32 = jnp.float32

N = 10000
E = 320000
F = 128
U = 128

NC = 2
NS = 16
NW = NC * NS
EPW = E // NW
H = 48
NH = 208
TAIL = EPW - NH * H
HG = H // 16
NP = 10240
DR = NP // 128
RPT = NP // NS
ZR = 128
KV = U // 16


def _tc_pre_body(node_ref, w_ref, b_ref, wn_ref, sin_ref, sout_ref):
    y = jnp.dot(node_ref[...], w_ref[...], preferred_element_type=F32)
    y = y + b_ref[...]
    wn_ref[...] = y[:, 0:U]
    sin_ref[...] = y[:, U:2 * U]
    sout_ref[...] = y[:, 2 * U:3 * U]


def _tc_pre(node, w_cat, b_cat):
    bn = 1000
    grid = (N // bn,)
    out = jax.ShapeDtypeStruct((N, U), F32)
    return pl.pallas_call(
        _tc_pre_body,
        grid=grid,
        in_specs=[
            pl.BlockSpec((bn, F), lambda i: (i, 0)),
            pl.BlockSpec((F, 3 * U), lambda i: (0, 0)),
            pl.BlockSpec((1, 3 * U), lambda i: (0, 0)),
        ],
        out_specs=[
            pl.BlockSpec((bn, U), lambda i: (i, 0)),
            pl.BlockSpec((bn, U), lambda i: (i, 0)),
            pl.BlockSpec((bn, U), lambda i: (i, 0)),
        ],
        out_shape=[out, out, out],
    )(node, w_cat, b_cat)


def _sc_body(sin_hbm, sout_hbm, wn_hbm, ii_hbm, io_hbm, av_hbm,
             num_out, den_out,
             av_v, ii0, ii1, io0, io1, iit, iot, sin0, sin1, sout0, sout1,
             w0, w1, ex0, ex1, dz_v,
             acc_sh, den_sh, is0, is1, gs0, gs1, gs2, gs3, gs4, gs5,
             ss0, ss1):
    ii_s = [ii0, ii1]
    io_s = [io0, io1]
    sin_s = [sin0, sin1]
    sout_s = [sout0, sout1]
    w_s = [w0, w1]
    ex_s = [ex0, ex1]
    isem = [is0, is1]
    gsem = [[gs0, gs1, gs2], [gs3, gs4, gs5]]
    ssem = [ss0, ss1]
    cid = lax.axis_index("c")
    sid = lax.axis_index("s")
    wid = cid * NS + sid
    z16 = jnp.zeros((16,), F32)
    iota16 = lax.iota(jnp.int32, 16)
    perms = [lax.bitwise_and(iota16 + sh, 15) for sh in (8, 4, 2, 1)]

    def _hsum(x):
        for p in perms:
            x = x + jnp.take_along_axis(x, p, axis=0)
        return x

    def _zden(r, _):
        dz_v[pl.ds(r * 16, 16)] = z16
        return _
    lax.fori_loop(0, RPT // 16, _zden, None)
    pltpu.sync_copy(dz_v, den_sh.at[pl.ds(sid * RPT, RPT)])

    def _zzb(r, _):
        for k in range(KV):
            sin0[r, pl.ds(k * 16, 16)] = z16
            sin1[r, pl.ds(k * 16, 16)] = z16
        return _
    lax.fori_loop(0, H, _zzb, None)
    zoff = sid * RPT
    for j in range(RPT // (2 * H)):
        pltpu.sync_copy(sin0, acc_sh.at[pl.ds(zoff + j * 2 * H, H)])
        pltpu.sync_copy(sin1, acc_sh.at[pl.ds(zoff + j * 2 * H + H, H)])
    pltpu.sync_copy(sin0.at[pl.ds(0, 40)],
                    acc_sh.at[pl.ds(zoff + RPT - 64, 40)])
    pltpu.sync_copy(sin1.at[pl.ds(0, 24)],
                    acc_sh.at[pl.ds(zoff + RPT - 24, 24)])
    plsc.subcore_barrier()

    pltpu.sync_copy(av_hbm, av_v)
    av = [av_v[pl.ds(k * 16, 16)] for k in range(KV)]

    def _start_idx(h, s):
        off = wid * EPW + h * H
        pltpu.async_copy(ii_hbm.at[pl.ds(off, H)], ii_s[s], isem[s])
        pltpu.async_copy(io_hbm.at[pl.ds(off, H)], io_s[s], isem[s])

    def _wait_idx(h, s):
        off = wid * EPW + h * H
        pltpu.make_async_copy(ii_hbm.at[pl.ds(off, H)], ii_s[s],
                              isem[s]).wait()
        pltpu.make_async_copy(io_hbm.at[pl.ds(off, H)], io_s[s],
                              isem[s]).wait()

    def _start_gath(s):
        pltpu.async_copy(sin_hbm.at[ii_s[s]], sin_s[s], gsem[s][0])
        pltpu.async_copy(sout_hbm.at[io_s[s]], sout_s[s], gsem[s][1])
        pltpu.async_copy(wn_hbm.at[io_s[s]], w_s[s], gsem[s][2])

    def _wait_gath(s):
        pltpu.make_async_copy(sin_hbm.at[ii_s[s]], sin_s[s],
                              gsem[s][0]).wait()
        pltpu.make_async_copy(sout_hbm.at[io_s[s]], sout_s[s],
                              gsem[s][1]).wait()
        pltpu.make_async_copy(wn_hbm.at[io_s[s]], w_s[s], gsem[s][2]).wait()

    def _score(sin_v, sout_v, ex_v, ngroups):
        def _group(g, _c):
            e0 = g * 16
            gv = z16
            for l in range(16):
                e = e0 + l
                acc = z16
                for k in range(KV):
                    tt = (sin_v[e, pl.ds(k * 16, 16)]
                          + sout_v[e, pl.ds(k * 16, 16)])
                    lr = jnp.maximum(tt, 0.2 * tt)
                    acc = acc + lr * av[k]
                gv = jnp.where(iota16 == l, _hsum(acc), gv)
            ex_v[pl.ds(e0, 16)] = jnp.exp(gv)
            return _c
        lax.fori_loop(0, ngroups, _group, None)

    def _scale(w_v, ex_v, ngroups):
        def _scaleg(g, _c):
            e0 = g * 16
            exg = ex_v[pl.ds(e0, 16)]
            for l in range(16):
                e = e0 + l
                ex = exg[l]
                for k in range(KV):
                    w_v[e, pl.ds(k * 16, 16)] = w_v[e, pl.ds(k * 16, 16)] * ex
            return _c
        lax.fori_loop(0, ngroups, _scaleg, None)

    def _process(s):
        _score(sin_s[s], sout_s[s], ex_s[s], HG)
        _scale(w_s[s], ex_s[s], HG)
        pltpu.async_copy(w_s[s], acc_sh.at[ii_s[s]], ssem[s], add=True)
        pltpu.async_copy(ex_s[s], den_sh.at[ii_s[s]], ssem[s], add=True)

    def _wait_scat(s):
        pltpu.make_async_copy(w_s[s], acc_sh.at[ii_s[s]], ssem[s]).wait()
        pltpu.make_async_copy(ex_s[s], den_sh.at[ii_s[s]], ssem[s]).wait()

    _start_idx(0, 0)
    _wait_idx(0, 0)
    _start_gath(0)
    _start_idx(1, 1)

    def _pair(m, _):
        h0 = 2 * m
        _wait_idx(h0 + 1, 1)

        @pl.when(m > 0)
        def _():
            _wait_scat(1)
        _wait_gath(0)
        _start_gath(1)
        _process(0)

        @pl.when(h0 + 2 < NH)
        def _():
            _start_idx(h0 + 2, 0)
            _wait_idx(h0 + 2, 0)
            _wait_scat(0)
            _start_gath(0)
        _wait_gath(1)
        _process(1)

        @pl.when(h0 + 3 < NH)
        def _():
            _start_idx(h0 + 3, 1)
        return _
    lax.fori_loop(0, NH // 2, _pair, None)
    _wait_scat(0)
    _wait_scat(1)

    toff = wid * EPW + NH * H
    pltpu.sync_copy(ii_hbm.at[pl.ds(toff, TAIL)], iit)
    pltpu.sync_copy(io_hbm.at[pl.ds(toff, TAIL)], iot)
    cpt1 = pltpu.async_copy(sin_hbm.at[iit], sin0.at[pl.ds(0, TAIL)], gs0)
    cpt2 = pltpu.async_copy(sout_hbm.at[iot], sout0.at[pl.ds(0, TAIL)], gs1)
    cpt3 = pltpu.async_copy(wn_hbm.at[iot], w0.at[pl.ds(0, TAIL)], gs2)
    cpt1.wait()
    cpt2.wait()
    cpt3.wait()
    _score(sin0, sout0, ex0, TAIL // 16)
    _scale(w0, ex0, TAIL // 16)
    pltpu.sync_copy(w0.at[pl.ds(0, TAIL)], acc_sh.at[iit], add=True)
    pltpu.sync_copy(ex0.at[pl.ds(0, TAIL)], den_sh.at[iit], add=True)

    plsc.subcore_barrier()

    for j in range(RPT // ZR):
        r0 = sid * RPT + j * ZR
        pltpu.sync_copy(acc_sh.at[pl.ds(r0, ZR)], num_out.at[cid, pl.ds(r0, ZR)])
    pltpu.sync_copy(den_sh.at[pl.ds(sid * RPT, RPT)],
                    den_out.at[pl.ds(cid * NP + sid * RPT, RPT)])


def _sc_call(sin, sout, wn, ii, io, av):
    mesh = plsc.VectorSubcoreMesh(core_axis_name="c", subcore_axis_name="s")
    kern = pl.kernel(
        _sc_body,
        out_type=(
            jax.ShapeDtypeStruct((NC, NP, U), F32),
            jax.ShapeDtypeStruct((NC * NP,), F32),
        ),
        mesh=mesh,
        scratch_types=[
            pltpu.VMEM((U,), F32),
            pltpu.VMEM((H,), jnp.int32),
            pltpu.VMEM((H,), jnp.int32),
            pltpu.VMEM((H,), jnp.int32),
            pltpu.VMEM((H,), jnp.int32),
            pltpu.VMEM((TAIL,), jnp.int32),
            pltpu.VMEM((TAIL,), jnp.int32),
            pltpu.VMEM((H, U), F32),
            pltpu.VMEM((H, U), F32),
            pltpu.VMEM((H, U), F32),
            pltpu.VMEM((H, U), F32),
            pltpu.VMEM((H, U), F32),
            pltpu.VMEM((H, U), F32),
            pltpu.VMEM((H,), F32),
            pltpu.VMEM((H,), F32),
            pltpu.VMEM((RPT,), F32),
            pltpu.VMEM_SHARED((NP, U), F32),
            pltpu.VMEM_SHARED((NP,), F32),
            pltpu.SemaphoreType.DMA,
            pltpu.SemaphoreType.DMA,
            pltpu.SemaphoreType.DMA,
            pltpu.SemaphoreType.DMA,
            pltpu.SemaphoreType.DMA,
            pltpu.SemaphoreType.DMA,
            pltpu.SemaphoreType.DMA,
            pltpu.SemaphoreType.DMA,
            pltpu.SemaphoreType.DMA,
            pltpu.SemaphoreType.DMA,
        ],
    )
    return kern(sin, sout, wn, ii, io, av)


def _tc_post_body(np_ref, dp_ref, out_ref):
    num = np_ref[0] + np_ref[1]
    den = dp_ref[0] + dp_ref[1]
    den = jnp.where(den > 0.0, den, 1.0)
    h = num[:N] / den
    out_ref[...] = jnp.maximum(h, 0.2 * h)


def _tc_post(num_parts, den_flat):
    return pl.pallas_call(
        _tc_post_body,
        out_shape=jax.ShapeDtypeStruct((N, U), F32),
    )(num_parts, den_flat)


def kernel(node, edge, edge_index, W_lin, b_lin, W_att, b_att, a_vec):
    del edge
    w_cat = jnp.concatenate([W_lin, W_att[:F], W_att[F:]], axis=1)
    b_cat = jnp.concatenate(
        [b_lin, b_att, jnp.zeros((U,), F32)]).reshape(1, 3 * U)
    wn, sin, sout = _tc_pre(node, w_cat, b_cat)
    ii = edge_index[0]
    io = edge_index[1]
    av = a_vec[:, 0]
    num_parts, den_parts = _sc_call(sin, sout, wn, ii, io, av)
    dp = den_parts.reshape(NC, NP)[:, :N, None]
    return _tc_post(num_parts, dp)
